# baseline (speedup 1.0000x reference)
import functools

import jax
import jax.numpy as jnp
from jax import lax
from jax.experimental import pallas as pl
from jax.experimental.pallas import tpu as pltpu
from jax.experimental.pallas import tpu_sc as plsc

BATCH = 16384
D = 64
L = 16


def _mf_call(src, dst, ue_flat, user_bias_flat, ie_flat, item_bias_flat, mean):
    info = plsc.get_sparse_core_info()
    nw = info.num_cores * info.num_subcores
    bw = BATCH // nw
    nblk = bw // L

    mesh = plsc.VectorSubcoreMesh(core_axis_name="c", subcore_axis_name="s")

    @functools.partial(
        pl.kernel,
        out_type=jax.ShapeDtypeStruct((BATCH,), jnp.float32),
        mesh=mesh,
        compiler_params=pltpu.CompilerParams(needs_layout_passes=False),
        scratch_types=[
            pltpu.VMEM((bw,), jnp.int32),
            pltpu.VMEM((bw,), jnp.int32),
            pltpu.VMEM((bw * D,), jnp.float32),
            pltpu.VMEM((bw * D,), jnp.float32),
            pltpu.VMEM((bw,), jnp.float32),
            pltpu.VMEM((bw,), jnp.float32),
            pltpu.VMEM((bw,), jnp.float32),
            pltpu.VMEM((L * L,), jnp.float32),
            pltpu.VMEM((L,), jnp.float32),
            pltpu.SemaphoreType.DMA,
            pltpu.SemaphoreType.DMA,
            pltpu.SemaphoreType.DMA,
            pltpu.SemaphoreType.DMA,
        ],
    )
    def mf_kernel(src_hbm, dst_hbm, ue_hbm, ub_hbm, ie_hbm, ib_hbm, mean_hbm,
                  out_hbm, sidx_v, didx_v, urows, vrows, ub_v, ib_v, out_v,
                  tscr, mean_v, sem_u, sem_v, sem_ub, sem_ib):
        wid = lax.axis_index("s") * info.num_cores + lax.axis_index("c")
        base = wid * bw

        pltpu.sync_copy(src_hbm.at[pl.ds(base, bw)], sidx_v)
        pltpu.sync_copy(dst_hbm.at[pl.ds(base, bw)], didx_v)
        cub = pltpu.async_copy(ub_hbm.at[sidx_v], ub_v, sem_ub)
        cib = pltpu.async_copy(ib_hbm.at[didx_v], ib_v, sem_ib)

        def dma_body(g, carry):
            off = g * L
            sv = sidx_v[pl.ds(off, L)]
            dv = didx_v[pl.ds(off, L)]
            for j in range(L):
                i = off + j
                pltpu.async_copy(ue_hbm.at[pl.ds(sv[j] * D, D)],
                                 urows.at[pl.ds(i * D, D)], sem_u)
                pltpu.async_copy(ie_hbm.at[pl.ds(dv[j] * D, D)],
                                 vrows.at[pl.ds(i * D, D)], sem_v)
            return carry

        lax.fori_loop(0, bw // L, dma_body, 0)

        pltpu.sync_copy(mean_hbm, mean_v)
        cub.wait()
        cib.wait()

        def wait_body(i, carry):
            pltpu.make_async_copy(ue_hbm.at[pl.ds(0, D)],
                                  urows.at[pl.ds(i * D, D)], sem_u).wait()
            pltpu.make_async_copy(ie_hbm.at[pl.ds(0, D)],
                                  vrows.at[pl.ds(i * D, D)], sem_v).wait()
            return carry

        lax.fori_loop(0, bw, wait_body, 0, unroll=4)

        m = mean_v[...]
        lane16 = jnp.arange(L, dtype=jnp.int32) * L

        def blk_body(b, carry):
            rb = b * L
            for r in range(L):
                row = (rb + r) * D
                s = urows[pl.ds(row, L)] * vrows[pl.ds(row, L)]
                for c in range(1, D // L):
                    s = s + (urows[pl.ds(row + c * L, L)]
                             * vrows[pl.ds(row + c * L, L)])
                plsc.store_scatter(tscr, [lane16 + r], s)
            acc = tscr[pl.ds(0, L)]
            for j in range(1, L):
                acc = acc + tscr[pl.ds(j * L, L)]
            out_v[pl.ds(rb, L)] = (
                acc + ub_v[pl.ds(rb, L)] + ib_v[pl.ds(rb, L)] + m)
            return carry

        lax.fori_loop(0, nblk, blk_body, 0)
        pltpu.sync_copy(out_v, out_hbm.at[pl.ds(base, bw)])

    return mf_kernel(src, dst, ue_flat, user_bias_flat, ie_flat,
                     item_bias_flat, mean)


def kernel(src, dst, user_emb, user_bias, item_emb, item_bias, mean):
    return _mf_call(
        src.astype(jnp.int32),
        dst.astype(jnp.int32),
        user_emb.reshape(-1),
        user_bias.reshape(-1),
        item_emb.reshape(-1),
        item_bias.reshape(-1),
        jnp.broadcast_to(mean, (L,)),
    )


# speedup vs baseline: 1.9919x; 1.9919x over previous
import functools

import jax
import jax.numpy as jnp
from jax import lax
from jax.experimental import pallas as pl
from jax.experimental.pallas import tpu as pltpu
from jax.experimental.pallas import tpu_sc as plsc

BATCH = 16384
V = 1000000
D = 64
L = 16
TAIL0 = (V // 128) * 128
NTAIL = V - TAIL0


def _stage_call(ueT, ieT, tail_u, tail_v, ssrc, sord, sdst, dord):
    info = plsc.get_sparse_core_info()
    nw = info.num_cores * info.num_subcores
    bw = BATCH // nw

    mesh = plsc.VectorSubcoreMesh(core_axis_name="c", subcore_axis_name="s")

    @functools.partial(
        pl.kernel,
        out_type=(jax.ShapeDtypeStruct((BATCH * D + D,), jnp.float32),
                  jax.ShapeDtypeStruct((BATCH * D + D,), jnp.float32)),
        mesh=mesh,
        compiler_params=pltpu.CompilerParams(
            needs_layout_passes=False, use_tc_tiling_on_sc=True),
        scratch_types=[
            pltpu.VMEM((bw,), jnp.int32),
            pltpu.VMEM((bw,), jnp.int32),
            pltpu.VMEM((D, 128), jnp.float32),
            pltpu.VMEM((NTAIL, D), jnp.float32),
            pltpu.VMEM((L * D,), jnp.float32),
            pltpu.SemaphoreType.DMA,
            pltpu.SemaphoreType.DMA,
            pltpu.SemaphoreType.DMA,
        ],
    )
    def k1(ueT_hbm, ieT_hbm, tu_hbm, tv_hbm, ssrc_hbm, sord_hbm, sdst_hbm,
           dord_hbm, su_hbm, sv_hbm, sx_v, sp_v, slab_v, tail_vm, rows_v,
           sem_s, sem_w, sem_t):
        wid = lax.axis_index("s") * info.num_cores + lax.axis_index("c")
        base = wid * bw
        lane16 = jnp.arange(L, dtype=jnp.int32)

        def stage_table(tab_hbm, tail_hbm, sid_hbm, pos_hbm, out_hbm):
            pltpu.sync_copy(sid_hbm.at[pl.ds(base, bw)], sx_v)
            pltpu.sync_copy(pos_hbm.at[pl.ds(base, bw)], sp_v)
            pltpu.async_copy(tail_hbm, tail_vm, sem_t).wait()
            for j in range(L):
                pltpu.async_copy(rows_v.at[pl.ds(j * D, D)],
                                 out_hbm.at[pl.ds(BATCH * D, D)], sem_w)

            def grp(g, cur):
                for j in range(L):
                    pltpu.make_async_copy(rows_v.at[pl.ds(j * D, D)],
                                          out_hbm.at[pl.ds(0, D)],
                                          sem_w).wait()
                xv = sx_v[pl.ds(g * L, L)]
                pv = sp_v[pl.ds(g * L, L)]
                for j in range(L):
                    x = xv[j]
                    p = pv[j]
                    s0 = (x >> 7) << 7
                    is_tail = s0 >= TAIL0
                    fetch = jnp.logical_and(s0 != cur, jnp.logical_not(is_tail))

                    @pl.when(fetch)
                    def _():
                        pltpu.async_copy(
                            tab_hbm.at[:, pl.ds(pl.multiple_of(s0, 128), 128)],
                            slab_v, sem_s).wait()

                    cur = jnp.where(is_tail, cur, s0)
                    lane = x - s0
                    tr = jnp.maximum(x - TAIL0, 0)
                    tsel = jnp.full((L,), 0, jnp.int32) + is_tail.astype(jnp.int32)
                    for c in range(D // L):
                        slab_vals = plsc.load_gather(
                            slab_v,
                            [lane16 + c * L,
                             jnp.full((L,), 0, jnp.int32) + lane])
                        tail_vals = tail_vm[tr, pl.ds(c * L, L)]
                        rows_v[pl.ds(j * D + c * L, L)] = jnp.where(
                            tsel > 0, tail_vals, slab_vals)

                    pltpu.async_copy(rows_v.at[pl.ds(j * D, D)],
                                     out_hbm.at[pl.ds(p * D, D)], sem_w)
                return cur

            lax.fori_loop(0, bw // L, grp, jnp.int32(-1))
            for j in range(L):
                pltpu.make_async_copy(rows_v.at[pl.ds(j * D, D)],
                                      out_hbm.at[pl.ds(0, D)], sem_w).wait()

        stage_table(ueT_hbm, tu_hbm, ssrc_hbm, sord_hbm, su_hbm)
        stage_table(ieT_hbm, tv_hbm, sdst_hbm, dord_hbm, sv_hbm)

    return k1(ueT, ieT, tail_u, tail_v, ssrc, sord, sdst, dord)


def _compute_call(stag_u, stag_v, src, dst, ub_flat, ib_flat, mean):
    info = plsc.get_sparse_core_info()
    nw = info.num_cores * info.num_subcores
    bw = BATCH // nw
    nblk = bw // L

    mesh = plsc.VectorSubcoreMesh(core_axis_name="c", subcore_axis_name="s")

    @functools.partial(
        pl.kernel,
        out_type=jax.ShapeDtypeStruct((BATCH,), jnp.float32),
        mesh=mesh,
        compiler_params=pltpu.CompilerParams(needs_layout_passes=False),
        scratch_types=[
            pltpu.VMEM((bw,), jnp.int32),
            pltpu.VMEM((bw,), jnp.int32),
            pltpu.VMEM((bw * D,), jnp.float32),
            pltpu.VMEM((bw * D,), jnp.float32),
            pltpu.VMEM((bw,), jnp.float32),
            pltpu.VMEM((bw,), jnp.float32),
            pltpu.VMEM((bw,), jnp.float32),
            pltpu.VMEM((L * L,), jnp.float32),
            pltpu.VMEM((L,), jnp.float32),
            pltpu.SemaphoreType.DMA,
            pltpu.SemaphoreType.DMA,
            pltpu.SemaphoreType.DMA,
            pltpu.SemaphoreType.DMA,
        ],
    )
    def k2(su_hbm, sv_hbm, src_hbm, dst_hbm, ub_hbm, ib_hbm, mean_hbm,
           out_hbm, sidx_v, didx_v, urows, vrows, ub_v, ib_v, out_v, tscr,
           mean_v, sem_u, sem_v, sem_ub, sem_ib):
        wid = lax.axis_index("s") * info.num_cores + lax.axis_index("c")
        base = wid * bw

        cu = pltpu.async_copy(su_hbm.at[pl.ds(base * D, bw * D)], urows,
                              sem_u)
        cv = pltpu.async_copy(sv_hbm.at[pl.ds(base * D, bw * D)], vrows,
                              sem_v)
        pltpu.sync_copy(src_hbm.at[pl.ds(base, bw)], sidx_v)
        pltpu.sync_copy(dst_hbm.at[pl.ds(base, bw)], didx_v)
        cub = pltpu.async_copy(ub_hbm.at[sidx_v], ub_v, sem_ub)
        cib = pltpu.async_copy(ib_hbm.at[didx_v], ib_v, sem_ib)
        pltpu.sync_copy(mean_hbm, mean_v)
        cu.wait()
        cv.wait()
        cub.wait()
        cib.wait()

        m = mean_v[...]
        lane16 = jnp.arange(L, dtype=jnp.int32) * L

        def blk_body(b, carry):
            rb = b * L
            for r in range(L):
                row = (rb + r) * D
                s = urows[pl.ds(row, L)] * vrows[pl.ds(row, L)]
                for c in range(1, D // L):
                    s = s + (urows[pl.ds(row + c * L, L)]
                             * vrows[pl.ds(row + c * L, L)])
                plsc.store_scatter(tscr, [lane16 + r], s)
            acc = tscr[pl.ds(0, L)]
            for j in range(1, L):
                acc = acc + tscr[pl.ds(j * L, L)]
            out_v[pl.ds(rb, L)] = (
                acc + ub_v[pl.ds(rb, L)] + ib_v[pl.ds(rb, L)] + m)
            return carry

        lax.fori_loop(0, nblk, blk_body, 0)
        pltpu.sync_copy(out_v, out_hbm.at[pl.ds(base, bw)])

    return k2(stag_u, stag_v, src, dst, ub_flat, ib_flat, mean)


def kernel(src, dst, user_emb, user_bias, item_emb, item_bias, mean):
    src = src.astype(jnp.int32)
    dst = dst.astype(jnp.int32)
    sord = jnp.argsort(src).astype(jnp.int32)
    dord = jnp.argsort(dst).astype(jnp.int32)
    ssrc = jnp.take(src, sord)
    sdst = jnp.take(dst, dord)
    tail_u = user_emb[TAIL0:, :]
    tail_v = item_emb[TAIL0:, :]
    stag_u, stag_v = _stage_call(user_emb.T, item_emb.T, tail_u, tail_v,
                                 ssrc, sord, sdst, dord)
    return _compute_call(stag_u, stag_v, src, dst,
                         user_bias.reshape(-1), item_bias.reshape(-1),
                         jnp.broadcast_to(mean, (L,)))


# speedup vs baseline: 2.3143x; 1.1619x over previous
import functools

import jax
import jax.numpy as jnp
from jax import lax
from jax.experimental import pallas as pl
from jax.experimental.pallas import tpu as pltpu
from jax.experimental.pallas import tpu_sc as plsc

BATCH = 16384
V = 1000000
D = 64
L = 16
TAIL0 = (V // 128) * 128
NTAIL = V - TAIL0


def _stage_call(ueT, ieT, tail_u, tail_v, ssrc, sord, sdst, dord):
    info = plsc.get_sparse_core_info()
    nw = info.num_cores * info.num_subcores
    bw = BATCH // nw

    mesh = plsc.VectorSubcoreMesh(core_axis_name="c", subcore_axis_name="s")

    @functools.partial(
        pl.kernel,
        out_type=(jax.ShapeDtypeStruct((BATCH * D + D,), jnp.float32),
                  jax.ShapeDtypeStruct((BATCH * D + D,), jnp.float32)),
        mesh=mesh,
        compiler_params=pltpu.CompilerParams(
            needs_layout_passes=False, use_tc_tiling_on_sc=True),
        scratch_types=[
            pltpu.VMEM((bw,), jnp.int32),
            pltpu.VMEM((bw,), jnp.int32),
            pltpu.VMEM((bw,), jnp.int32),
            pltpu.VMEM((bw,), jnp.int32),
            pltpu.VMEM((D, 128), jnp.float32),
            pltpu.VMEM((D, 128), jnp.float32),
            pltpu.VMEM((NTAIL, D), jnp.float32),
            pltpu.VMEM((NTAIL, D), jnp.float32),
            pltpu.VMEM((L * D,), jnp.float32),
            pltpu.VMEM((L * D,), jnp.float32),
            pltpu.SemaphoreType.DMA,
            pltpu.SemaphoreType.DMA,
            pltpu.SemaphoreType.DMA,
            pltpu.SemaphoreType.DMA,
            pltpu.SemaphoreType.DMA,
        ],
    )
    def k1(ueT_hbm, ieT_hbm, tu_hbm, tv_hbm, ssrc_hbm, sord_hbm, sdst_hbm,
           dord_hbm, su_hbm, sv_hbm, ux_v, up_v, ix_v, ip_v, uslab_v, islab_v,
           utail_vm, itail_vm, urow_v, irow_v, sem_us, sem_is, sem_uw, sem_iw,
           sem_t):
        wid = lax.axis_index("s") * info.num_cores + lax.axis_index("c")
        base = wid * bw
        lane16 = jnp.arange(L, dtype=jnp.int32)

        pltpu.sync_copy(ssrc_hbm.at[pl.ds(base, bw)], ux_v)
        pltpu.sync_copy(sord_hbm.at[pl.ds(base, bw)], up_v)
        pltpu.sync_copy(sdst_hbm.at[pl.ds(base, bw)], ix_v)
        pltpu.sync_copy(dord_hbm.at[pl.ds(base, bw)], ip_v)
        pltpu.async_copy(tu_hbm, utail_vm, sem_t).wait()
        pltpu.async_copy(tv_hbm, itail_vm, sem_t).wait()
        for j in range(L):
            pltpu.async_copy(urow_v.at[pl.ds(j * D, D)],
                             su_hbm.at[pl.ds(BATCH * D, D)], sem_uw)
            pltpu.async_copy(irow_v.at[pl.ds(j * D, D)],
                             sv_hbm.at[pl.ds(BATCH * D, D)], sem_iw)

        def fetch_slab(x, cur, tab_hbm, slab, sem):
            s0 = (x >> 7) << 7
            is_tail = s0 >= TAIL0
            fetch = jnp.logical_and(s0 != cur, jnp.logical_not(is_tail))

            @pl.when(fetch)
            def _():
                pltpu.async_copy(
                    tab_hbm.at[:, pl.ds(pl.multiple_of(s0, 128), 128)],
                    slab, sem)

            return jnp.where(is_tail, cur, s0), fetch, is_tail, x - s0

        def extract(x, is_tail, lane, slab, tail_vm, rows, j):
            tr = jnp.maximum(x - TAIL0, 0)
            tsel = jnp.full((L,), 0, jnp.int32) + is_tail.astype(jnp.int32)
            for c in range(D // L):
                slab_vals = plsc.load_gather(
                    slab, [lane16 + c * L,
                           jnp.full((L,), 0, jnp.int32) + lane])
                tail_vals = tail_vm[tr, pl.ds(c * L, L)]
                rows[pl.ds(j * D + c * L, L)] = jnp.where(
                    tsel > 0, tail_vals, slab_vals)

        def grp(g, carry):
            cur_u, cur_i = carry
            for j in range(L):
                pltpu.make_async_copy(urow_v.at[pl.ds(j * D, D)],
                                      su_hbm.at[pl.ds(0, D)], sem_uw).wait()
                pltpu.make_async_copy(irow_v.at[pl.ds(j * D, D)],
                                      sv_hbm.at[pl.ds(0, D)], sem_iw).wait()
            xu = ux_v[pl.ds(g * L, L)]
            pu = up_v[pl.ds(g * L, L)]
            xi = ix_v[pl.ds(g * L, L)]
            pi = ip_v[pl.ds(g * L, L)]
            for j in range(L):
                u = xu[j]
                i = xi[j]
                cur_u, fu, ut, ulane = fetch_slab(u, cur_u, ueT_hbm,
                                                  uslab_v, sem_us)
                cur_i, fi, it, ilane = fetch_slab(i, cur_i, ieT_hbm,
                                                  islab_v, sem_is)

                @pl.when(fu)
                def _():
                    pltpu.make_async_copy(ueT_hbm.at[:, pl.ds(0, 128)],
                                          uslab_v, sem_us).wait()

                extract(u, ut, ulane, uslab_v, utail_vm, urow_v, j)
                pltpu.async_copy(urow_v.at[pl.ds(j * D, D)],
                                 su_hbm.at[pl.ds(pu[j] * D, D)], sem_uw)

                @pl.when(fi)
                def _():
                    pltpu.make_async_copy(ieT_hbm.at[:, pl.ds(0, 128)],
                                          islab_v, sem_is).wait()

                extract(i, it, ilane, islab_v, itail_vm, irow_v, j)
                pltpu.async_copy(irow_v.at[pl.ds(j * D, D)],
                                 sv_hbm.at[pl.ds(pi[j] * D, D)], sem_iw)
            return (cur_u, cur_i)

        lax.fori_loop(0, bw // L, grp, (jnp.int32(-1), jnp.int32(-1)))
        for j in range(L):
            pltpu.make_async_copy(urow_v.at[pl.ds(j * D, D)],
                                  su_hbm.at[pl.ds(0, D)], sem_uw).wait()
            pltpu.make_async_copy(irow_v.at[pl.ds(j * D, D)],
                                  sv_hbm.at[pl.ds(0, D)], sem_iw).wait()

    return k1(ueT, ieT, tail_u, tail_v, ssrc, sord, sdst, dord)


def _compute_call(stag_u, stag_v, src, dst, ub_flat, ib_flat, mean):
    info = plsc.get_sparse_core_info()
    nw = info.num_cores * info.num_subcores
    bw = BATCH // nw
    nblk = bw // L

    mesh = plsc.VectorSubcoreMesh(core_axis_name="c", subcore_axis_name="s")

    @functools.partial(
        pl.kernel,
        out_type=jax.ShapeDtypeStruct((BATCH,), jnp.float32),
        mesh=mesh,
        compiler_params=pltpu.CompilerParams(needs_layout_passes=False),
        scratch_types=[
            pltpu.VMEM((bw,), jnp.int32),
            pltpu.VMEM((bw,), jnp.int32),
            pltpu.VMEM((bw * D,), jnp.float32),
            pltpu.VMEM((bw * D,), jnp.float32),
            pltpu.VMEM((bw,), jnp.float32),
            pltpu.VMEM((bw,), jnp.float32),
            pltpu.VMEM((bw,), jnp.float32),
            pltpu.VMEM((L * L,), jnp.float32),
            pltpu.VMEM((L,), jnp.float32),
            pltpu.SemaphoreType.DMA,
            pltpu.SemaphoreType.DMA,
            pltpu.SemaphoreType.DMA,
            pltpu.SemaphoreType.DMA,
        ],
    )
    def k2(su_hbm, sv_hbm, src_hbm, dst_hbm, ub_hbm, ib_hbm, mean_hbm,
           out_hbm, sidx_v, didx_v, urows, vrows, ub_v, ib_v, out_v, tscr,
           mean_v, sem_u, sem_v, sem_ub, sem_ib):
        wid = lax.axis_index("s") * info.num_cores + lax.axis_index("c")
        base = wid * bw

        cu = pltpu.async_copy(su_hbm.at[pl.ds(base * D, bw * D)], urows,
                              sem_u)
        cv = pltpu.async_copy(sv_hbm.at[pl.ds(base * D, bw * D)], vrows,
                              sem_v)
        pltpu.sync_copy(src_hbm.at[pl.ds(base, bw)], sidx_v)
        pltpu.sync_copy(dst_hbm.at[pl.ds(base, bw)], didx_v)
        cub = pltpu.async_copy(ub_hbm.at[sidx_v], ub_v, sem_ub)
        cib = pltpu.async_copy(ib_hbm.at[didx_v], ib_v, sem_ib)
        pltpu.sync_copy(mean_hbm, mean_v)
        cu.wait()
        cv.wait()
        cub.wait()
        cib.wait()

        m = mean_v[...]
        lane16 = jnp.arange(L, dtype=jnp.int32) * L

        def blk_body(b, carry):
            rb = b * L
            for r in range(L):
                row = (rb + r) * D
                s = urows[pl.ds(row, L)] * vrows[pl.ds(row, L)]
                for c in range(1, D // L):
                    s = s + (urows[pl.ds(row + c * L, L)]
                             * vrows[pl.ds(row + c * L, L)])
                plsc.store_scatter(tscr, [lane16 + r], s)
            acc = tscr[pl.ds(0, L)]
            for j in range(1, L):
                acc = acc + tscr[pl.ds(j * L, L)]
            out_v[pl.ds(rb, L)] = (
                acc + ub_v[pl.ds(rb, L)] + ib_v[pl.ds(rb, L)] + m)
            return carry

        lax.fori_loop(0, nblk, blk_body, 0)
        pltpu.sync_copy(out_v, out_hbm.at[pl.ds(base, bw)])

    return k2(stag_u, stag_v, src, dst, ub_flat, ib_flat, mean)


def kernel(src, dst, user_emb, user_bias, item_emb, item_bias, mean):
    src = src.astype(jnp.int32)
    dst = dst.astype(jnp.int32)
    sord = jnp.argsort(src).astype(jnp.int32)
    dord = jnp.argsort(dst).astype(jnp.int32)
    ssrc = jnp.take(src, sord)
    sdst = jnp.take(dst, dord)
    tail_u = user_emb[TAIL0:, :]
    tail_v = item_emb[TAIL0:, :]
    stag_u, stag_v = _stage_call(user_emb.T, item_emb.T, tail_u, tail_v,
                                 ssrc, sord, sdst, dord)
    return _compute_call(stag_u, stag_v, src, dst,
                         user_bias.reshape(-1), item_bias.reshape(-1),
                         jnp.broadcast_to(mean, (L,)))


# speedup vs baseline: 2.7151x; 1.1731x over previous
import functools

import jax
import jax.numpy as jnp
from jax import lax
from jax.experimental import pallas as pl
from jax.experimental.pallas import tpu as pltpu
from jax.experimental.pallas import tpu_sc as plsc

BATCH = 16384
V = 1000000
D = 64
L = 16
TAIL0 = (V // 128) * 128
NTAIL = V - TAIL0


def _stage_call(ueT, ieT, tail_u, tail_v, ssrc, sord, sdst, dord):
    info = plsc.get_sparse_core_info()
    nw = info.num_cores * info.num_subcores
    bw = BATCH // nw

    mesh = plsc.VectorSubcoreMesh(core_axis_name="c", subcore_axis_name="s")

    @functools.partial(
        pl.kernel,
        out_type=(jax.ShapeDtypeStruct((BATCH * D + D,), jnp.float32),
                  jax.ShapeDtypeStruct((BATCH * D + D,), jnp.float32)),
        mesh=mesh,
        compiler_params=pltpu.CompilerParams(
            needs_layout_passes=False, use_tc_tiling_on_sc=True),
        scratch_types=[
            pltpu.VMEM((bw,), jnp.int32),
            pltpu.VMEM((bw,), jnp.int32),
            pltpu.VMEM((bw,), jnp.int32),
            pltpu.VMEM((bw,), jnp.int32),
            [pltpu.VMEM((D, 128), jnp.float32) for _ in range(4)],
            pltpu.VMEM((NTAIL, D), jnp.float32),
            pltpu.VMEM((NTAIL, D), jnp.float32),
            [pltpu.VMEM((L * D,), jnp.float32) for _ in range(4)],
            [pltpu.SemaphoreType.DMA for _ in range(4)],
            [pltpu.SemaphoreType.DMA for _ in range(4)],
            pltpu.SemaphoreType.DMA,
        ],
    )
    def k1(ueT_hbm, ieT_hbm, tu_hbm, tv_hbm, ssrc_hbm, sord_hbm, sdst_hbm,
           dord_hbm, su_hbm, sv_hbm, ux_v, up_v, ix_v, ip_v, slabs,
           utail_vm, itail_vm, rows, sems_s, sems_w, sem_t):
        wid = lax.axis_index("s") * info.num_cores + lax.axis_index("c")
        base = wid * bw
        lane16 = jnp.arange(L, dtype=jnp.int32)

        pltpu.sync_copy(ssrc_hbm.at[pl.ds(base, bw)], ux_v)
        pltpu.sync_copy(sord_hbm.at[pl.ds(base, bw)], up_v)
        pltpu.sync_copy(sdst_hbm.at[pl.ds(base, bw)], ix_v)
        pltpu.sync_copy(dord_hbm.at[pl.ds(base, bw)], ip_v)
        pltpu.async_copy(tu_hbm, utail_vm, sem_t).wait()
        pltpu.async_copy(tv_hbm, itail_vm, sem_t).wait()
        half = bw // 2
        streams = [
            (ueT_hbm, utail_vm, ux_v, up_v, su_hbm, 0),
            (ieT_hbm, itail_vm, ix_v, ip_v, sv_hbm, 0),
            (ueT_hbm, utail_vm, ux_v, up_v, su_hbm, half),
            (ieT_hbm, itail_vm, ix_v, ip_v, sv_hbm, half),
        ]
        for j in range(L):
            for k in range(4):
                pltpu.async_copy(rows[k].at[pl.ds(j * D, D)],
                                 streams[k][4].at[pl.ds(BATCH * D, D)],
                                 sems_w[k])

        def fetch_slab(x, cur, tab_hbm, slab, sem):
            s0 = (x >> 7) << 7
            is_tail = s0 >= TAIL0
            fetch = jnp.logical_and(s0 != cur, jnp.logical_not(is_tail))

            @pl.when(fetch)
            def _():
                pltpu.async_copy(
                    tab_hbm.at[:, pl.ds(pl.multiple_of(s0, 128), 128)],
                    slab, sem)

            return jnp.where(is_tail, cur, s0), fetch, is_tail, x - s0

        def extract(x, is_tail, lane, slab, tail_vm, rows, j):
            tr = jnp.maximum(x - TAIL0, 0)
            tsel = jnp.full((L,), 0, jnp.int32) + is_tail.astype(jnp.int32)
            for c in range(D // L):
                slab_vals = plsc.load_gather(
                    slab, [lane16 + c * L,
                           jnp.full((L,), 0, jnp.int32) + lane])
                tail_vals = tail_vm[tr, pl.ds(c * L, L)]
                rows[pl.ds(j * D + c * L, L)] = jnp.where(
                    tsel > 0, tail_vals, slab_vals)

        def grp(g, carry):
            curs = list(carry)
            for j in range(L):
                for k in range(4):
                    pltpu.make_async_copy(rows[k].at[pl.ds(j * D, D)],
                                          streams[k][4].at[pl.ds(0, D)],
                                          sems_w[k]).wait()
            xv = [streams[k][2][pl.ds(streams[k][5] + g * L, L)]
                  for k in range(4)]
            pv = [streams[k][3][pl.ds(streams[k][5] + g * L, L)]
                  for k in range(4)]
            for j in range(L):
                ent = []
                for k in range(4):
                    x = xv[k][j]
                    curs[k], fk, tk, lk = fetch_slab(
                        x, curs[k], streams[k][0], slabs[k], sems_s[k])
                    ent.append((x, fk, tk, lk))
                for k in range(4):
                    x, fk, tk, lk = ent[k]

                    @pl.when(fk)
                    def _(k=k):
                        pltpu.make_async_copy(
                            streams[k][0].at[:, pl.ds(0, 128)], slabs[k],
                            sems_s[k]).wait()

                    extract(x, tk, lk, slabs[k], streams[k][1], rows[k], j)
                    pltpu.async_copy(
                        rows[k].at[pl.ds(j * D, D)],
                        streams[k][4].at[pl.ds(pv[k][j] * D, D)], sems_w[k])
            return tuple(curs)

        lax.fori_loop(0, half // L, grp,
                      (jnp.int32(-1),) * 4)
        for j in range(L):
            for k in range(4):
                pltpu.make_async_copy(rows[k].at[pl.ds(j * D, D)],
                                      streams[k][4].at[pl.ds(0, D)],
                                      sems_w[k]).wait()

    return k1(ueT, ieT, tail_u, tail_v, ssrc, sord, sdst, dord)


def _compute_call(stag_u, stag_v, src, dst, ub_flat, ib_flat, mean):
    info = plsc.get_sparse_core_info()
    nw = info.num_cores * info.num_subcores
    bw = BATCH // nw
    nblk = bw // L

    mesh = plsc.VectorSubcoreMesh(core_axis_name="c", subcore_axis_name="s")

    @functools.partial(
        pl.kernel,
        out_type=jax.ShapeDtypeStruct((BATCH,), jnp.float32),
        mesh=mesh,
        compiler_params=pltpu.CompilerParams(needs_layout_passes=False),
        scratch_types=[
            pltpu.VMEM((bw,), jnp.int32),
            pltpu.VMEM((bw,), jnp.int32),
            pltpu.VMEM((bw * D,), jnp.float32),
            pltpu.VMEM((bw * D,), jnp.float32),
            pltpu.VMEM((bw,), jnp.float32),
            pltpu.VMEM((bw,), jnp.float32),
            pltpu.VMEM((bw,), jnp.float32),
            pltpu.VMEM((L * L,), jnp.float32),
            pltpu.VMEM((L,), jnp.float32),
            pltpu.SemaphoreType.DMA,
            pltpu.SemaphoreType.DMA,
            pltpu.SemaphoreType.DMA,
            pltpu.SemaphoreType.DMA,
        ],
    )
    def k2(su_hbm, sv_hbm, src_hbm, dst_hbm, ub_hbm, ib_hbm, mean_hbm,
           out_hbm, sidx_v, didx_v, urows, vrows, ub_v, ib_v, out_v, tscr,
           mean_v, sem_u, sem_v, sem_ub, sem_ib):
        wid = lax.axis_index("s") * info.num_cores + lax.axis_index("c")
        base = wid * bw

        cu = pltpu.async_copy(su_hbm.at[pl.ds(base * D, bw * D)], urows,
                              sem_u)
        cv = pltpu.async_copy(sv_hbm.at[pl.ds(base * D, bw * D)], vrows,
                              sem_v)
        pltpu.sync_copy(src_hbm.at[pl.ds(base, bw)], sidx_v)
        pltpu.sync_copy(dst_hbm.at[pl.ds(base, bw)], didx_v)
        cub = pltpu.async_copy(ub_hbm.at[sidx_v], ub_v, sem_ub)
        cib = pltpu.async_copy(ib_hbm.at[didx_v], ib_v, sem_ib)
        pltpu.sync_copy(mean_hbm, mean_v)
        cu.wait()
        cv.wait()
        cub.wait()
        cib.wait()

        m = mean_v[...]
        lane16 = jnp.arange(L, dtype=jnp.int32) * L

        def blk_body(b, carry):
            rb = b * L
            for r in range(L):
                row = (rb + r) * D
                s = urows[pl.ds(row, L)] * vrows[pl.ds(row, L)]
                for c in range(1, D // L):
                    s = s + (urows[pl.ds(row + c * L, L)]
                             * vrows[pl.ds(row + c * L, L)])
                plsc.store_scatter(tscr, [lane16 + r], s)
            acc = tscr[pl.ds(0, L)]
            for j in range(1, L):
                acc = acc + tscr[pl.ds(j * L, L)]
            out_v[pl.ds(rb, L)] = (
                acc + ub_v[pl.ds(rb, L)] + ib_v[pl.ds(rb, L)] + m)
            return carry

        lax.fori_loop(0, nblk, blk_body, 0)
        pltpu.sync_copy(out_v, out_hbm.at[pl.ds(base, bw)])

    return k2(stag_u, stag_v, src, dst, ub_flat, ib_flat, mean)


def kernel(src, dst, user_emb, user_bias, item_emb, item_bias, mean):
    src = src.astype(jnp.int32)
    dst = dst.astype(jnp.int32)
    sord = jnp.argsort(src).astype(jnp.int32)
    dord = jnp.argsort(dst).astype(jnp.int32)
    ssrc = jnp.take(src, sord)
    sdst = jnp.take(dst, dord)
    tail_u = user_emb[TAIL0:, :]
    tail_v = item_emb[TAIL0:, :]
    stag_u, stag_v = _stage_call(user_emb.T, item_emb.T, tail_u, tail_v,
                                 ssrc, sord, sdst, dord)
    return _compute_call(stag_u, stag_v, src, dst,
                         user_bias.reshape(-1), item_bias.reshape(-1),
                         jnp.broadcast_to(mean, (L,)))
